# baseline (speedup 1.0000x reference)
import functools

import jax
import jax.numpy as jnp
from jax import lax
from jax.experimental import pallas as pl
from jax.experimental.pallas import tpu as pltpu
from jax.experimental.pallas import tpu_sc as plsc

S, D = 2048, 768
E, K, H = 8, 2, 1536
P = K * S
BK = 256
NB = P // BK + (E - 1)
NBK = NB * BK
NC, NS = 2, 16
NW = NC * NS
RPT = NBK // NW
C1, C2 = 96, 88
TPB = S // NW
CH = 128


def _router_kernel(x_ref, wg_ref, bg_ref, gate_ref, pos_ref, qpair_ref,
                   beid_ref, oh_ref):
    xv = x_ref[...]
    logits = jnp.dot(xv, wg_ref[...], preferred_element_type=jnp.float32)
    logits = logits + bg_ref[...]
    m = jnp.max(logits, axis=1, keepdims=True)
    ex = jnp.exp(logits - m)
    probs = ex / jnp.sum(ex, axis=1, keepdims=True)
    gate_ref[...] = probs

    idx = lax.broadcasted_iota(jnp.int32, (S, E), 1)
    p1 = jnp.max(probs, axis=1, keepdims=True)
    i1 = jnp.min(jnp.where(probs == p1, idx, E), axis=1, keepdims=True)
    pm = jnp.where(idx == i1, -jnp.inf, probs)
    p2 = jnp.max(pm, axis=1, keepdims=True)
    i2 = jnp.min(jnp.where(pm == p2, idx, E), axis=1, keepdims=True)
    t = jnp.exp(p2 - p1)
    q1 = 1.0 / (1.0 + t)
    q2 = t / (1.0 + t)

    i12 = jnp.concatenate([i1, i2], axis=0)
    qpair_ref[...] = jnp.concatenate([q1, q2], axis=0)
    lane8 = lax.broadcasted_iota(jnp.int32, (P, E), 1)
    oh = (i12 == lane8).astype(jnp.float32)
    oh_ref[...] = oh

    cnt = jnp.sum(oh, axis=0, keepdims=True)
    npad = jnp.ceil(cnt / BK) * BK
    r8 = lax.broadcasted_iota(jnp.int32, (E, E), 0)
    c8 = lax.broadcasted_iota(jnp.int32, (E, E), 1)
    excl8 = (r8 < c8).astype(jnp.float32)
    poff = jnp.dot(npad, excl8, preferred_element_type=jnp.float32)

    rr = lax.broadcasted_iota(jnp.int32, (CH, CH), 0)
    cc = lax.broadcasted_iota(jnp.int32, (CH, CH), 1)
    ltri = (rr > cc).astype(jnp.float32)

    def body(i, base):
        blk = oh_ref[pl.ds(i * CH, CH), :]
        rank = jnp.dot(ltri, blk, preferred_element_type=jnp.float32)
        row = jnp.sum((rank + base + poff) * blk, axis=1, keepdims=True)
        pos_ref[pl.ds(i * CH, CH), :] = row.astype(jnp.int32)
        return base + jnp.sum(blk, axis=0, keepdims=True)

    lax.fori_loop(0, P // CH, body, jnp.zeros((1, E), jnp.float32))

    nblk = npad / BK
    blkoff = jnp.dot(nblk, excl8, preferred_element_type=jnp.float32)
    biota = lax.broadcasted_iota(jnp.int32, (32, E), 0).astype(jnp.float32)
    ge = (biota >= blkoff).astype(jnp.float32)
    beid_ref[...] = (jnp.sum(ge, axis=1, keepdims=True) - 1.0).astype(jnp.int32)


def _sc_dispatch(pos_hbm, q_hbm, x_hbm, xg_hbm, wr_hbm,
                 pos_v, q_v, rt_v, wr_v, idx_v, buf, rt_sh, sem):
    cid = lax.axis_index("c")
    sid = lax.axis_index("s")

    @pl.when(sid == 0)
    def _scatter():
        pltpu.sync_copy(pos_hbm, pos_v)
        pltpu.sync_copy(q_hbm, q_v)
        zi = jnp.zeros((16,), jnp.int32)
        zf = jnp.zeros((16,), jnp.float32)

        def zbody(i, _):
            rt_v[pl.ds(i * 16, 16)] = zi
            wr_v[pl.ds(i * 16, 16)] = zf
            return 0

        lax.fori_loop(0, NBK // 16, zbody, 0)

        def sbody(i, _):
            pv = pos_v[pl.ds(i * 16, 16)]
            qv = q_v[pl.ds(i * 16, 16)]
            tok = (lax.iota(jnp.int32, 16) + i * 16) & (S - 1)
            plsc.store_scatter(rt_v, [pv], tok)
            plsc.store_scatter(wr_v, [pv], qv)
            return 0

        lax.fori_loop(0, P // 16, sbody, 0)
        pltpu.sync_copy(rt_v, rt_sh)

        @pl.when(cid == 0)
        def _():
            pltpu.sync_copy(wr_v, wr_hbm)

    plsc.subcore_barrier()
    tid = cid * NS + sid
    base = tid * RPT
    pltpu.sync_copy(rt_sh.at[pl.ds(base, RPT)], idx_v)
    pltpu.async_copy(x_hbm.at[idx_v.at[pl.ds(0, C1)]], buf, sem).wait()
    pltpu.sync_copy(buf, xg_hbm.at[pl.ds(base, C1)])
    pltpu.async_copy(x_hbm.at[idx_v.at[pl.ds(C1, C2)]],
                     buf.at[pl.ds(0, C2)], sem).wait()
    pltpu.sync_copy(buf.at[pl.ds(0, C2)], xg_hbm.at[pl.ds(base + C1, C2)])


def _gmm_kernel(beid_ref, xg_ref, w1_ref, b1_ref, w2_ref, b2_ref, wr_ref,
                y_ref):
    del beid_ref
    xv = xg_ref[...]
    acc = jnp.zeros((BK, D), jnp.float32)
    for hc in range(2):
        sl = slice(hc * (H // 2), (hc + 1) * (H // 2))
        h = jnp.maximum(
            jnp.dot(xv, w1_ref[0, :, sl], preferred_element_type=jnp.float32)
            + b1_ref[0, :, sl], 0.0)
        acc = acc + jnp.dot(h, w2_ref[0, sl, :],
                            preferred_element_type=jnp.float32)
    y_ref[...] = (acc + b2_ref[0]) * wr_ref[...]


def _sc_combine(pos_hbm, y_hbm, out_hbm, idx1_v, idx2_v, buf1, buf2, sem):
    cid = lax.axis_index("c")
    sid = lax.axis_index("s")
    tid = cid * NS + sid
    tb = tid * TPB
    pltpu.sync_copy(pos_hbm.at[pl.ds(tb, TPB)], idx1_v)
    pltpu.sync_copy(pos_hbm.at[pl.ds(S + tb, TPB)], idx2_v)
    c1 = pltpu.async_copy(y_hbm.at[idx1_v], buf1, sem)
    c2 = pltpu.async_copy(y_hbm.at[idx2_v], buf2, sem)
    c1.wait()
    c2.wait()

    def rbody(r, _):
        for c in range(D // 16):
            sl = pl.ds(c * 16, 16)
            buf1[r, sl] = buf1[r, sl] + buf2[r, sl]
        return 0

    lax.fori_loop(0, TPB, rbody, 0)
    pltpu.sync_copy(buf1, out_hbm.at[pl.ds(tb, TPB)])


@functools.lru_cache(maxsize=None)
def _sc_calls():
    mesh = plsc.VectorSubcoreMesh(core_axis_name="c", subcore_axis_name="s",
                                  num_cores=NC, num_subcores=NS)
    dispatch = pl.kernel(
        _sc_dispatch,
        out_type=[jax.ShapeDtypeStruct((NBK, D), jnp.float32),
                  jax.ShapeDtypeStruct((NBK,), jnp.float32)],
        mesh=mesh,
        scratch_types=[
            pltpu.VMEM((P,), jnp.int32),
            pltpu.VMEM((P,), jnp.float32),
            pltpu.VMEM((NBK,), jnp.int32),
            pltpu.VMEM((NBK,), jnp.float32),
            pltpu.VMEM((RPT,), jnp.int32),
            pltpu.VMEM((C1, D), jnp.float32),
            pltpu.VMEM_SHARED((NBK,), jnp.int32),
            pltpu.SemaphoreType.DMA,
        ],
        compiler_params=pltpu.CompilerParams(needs_layout_passes=False),
    )
    combine = pl.kernel(
        _sc_combine,
        out_type=jax.ShapeDtypeStruct((S, D), jnp.float32),
        mesh=mesh,
        scratch_types=[
            pltpu.VMEM((TPB,), jnp.int32),
            pltpu.VMEM((TPB,), jnp.int32),
            pltpu.VMEM((TPB, D), jnp.float32),
            pltpu.VMEM((TPB, D), jnp.float32),
            pltpu.SemaphoreType.DMA,
        ],
        compiler_params=pltpu.CompilerParams(needs_layout_passes=False),
    )
    return dispatch, combine


@jax.jit
def kernel(x, Wg, bg, W1, b1, W2, b2):
    B = x.shape[0]
    xs = x.reshape(S, D)

    gate, pos, qpair, beid = pl.pallas_call(
        _router_kernel,
        out_shape=[
            jax.ShapeDtypeStruct((S, E), jnp.float32),
            jax.ShapeDtypeStruct((P, 1), jnp.int32),
            jax.ShapeDtypeStruct((P, 1), jnp.float32),
            jax.ShapeDtypeStruct((32, 1), jnp.int32),
        ],
        scratch_shapes=[pltpu.VMEM((P, E), jnp.float32)],
    )(xs, Wg, bg.reshape(1, E))

    dispatch_call, combine_call = _sc_calls()
    pos1 = pos.reshape(P)
    xg, wrow = dispatch_call(pos1, qpair.reshape(P), xs)

    y = pl.pallas_call(
        _gmm_kernel,
        grid_spec=pltpu.PrefetchScalarGridSpec(
            num_scalar_prefetch=1,
            grid=(NB,),
            in_specs=[
                pl.BlockSpec((BK, D), lambda i, beid: (i, 0)),
                pl.BlockSpec((1, D, H), lambda i, beid: (beid[i], 0, 0)),
                pl.BlockSpec((1, 1, H), lambda i, beid: (beid[i], 0, 0)),
                pl.BlockSpec((1, H, D), lambda i, beid: (beid[i], 0, 0)),
                pl.BlockSpec((1, 1, D), lambda i, beid: (beid[i], 0, 0)),
                pl.BlockSpec((BK, 1), lambda i, beid: (i, 0)),
            ],
            out_specs=pl.BlockSpec((BK, D), lambda i, beid: (i, 0)),
        ),
        out_shape=jax.ShapeDtypeStruct((NBK, D), jnp.float32),
    )(beid.reshape(32), xg, W1, b1.reshape(E, 1, H), W2, b2.reshape(E, 1, D),
      wrow.reshape(NBK, 1))

    out = combine_call(pos1, y)
    return out.reshape(B, S, D), gate.reshape(B, S, E)


# speedup vs baseline: 1.0074x; 1.0074x over previous
import functools

import jax
import jax.numpy as jnp
from jax import lax
from jax.experimental import pallas as pl
from jax.experimental.pallas import tpu as pltpu
from jax.experimental.pallas import tpu_sc as plsc

S, D = 2048, 768
E, K, H = 8, 2, 1536
P = K * S
BK = 256
NB = P // BK + (E - 1)
NBK = NB * BK
NC, NS = 2, 16
NW = NC * NS
RPT = NBK // NW
C1, C2 = 96, 88
TPB = S // NW
CH = 128


def _router_kernel(x_ref, wg_ref, bg_ref, gate_ref, pos_ref, qpair_ref,
                   beid_ref, oh_ref):
    xv = x_ref[...]
    logits = jnp.dot(xv, wg_ref[...], preferred_element_type=jnp.float32)
    logits = logits + bg_ref[...]
    m = jnp.max(logits, axis=1, keepdims=True)
    ex = jnp.exp(logits - m)
    probs = ex / jnp.sum(ex, axis=1, keepdims=True)
    gate_ref[...] = probs

    idx = lax.broadcasted_iota(jnp.int32, (S, E), 1)
    p1 = jnp.max(probs, axis=1, keepdims=True)
    i1 = jnp.min(jnp.where(probs == p1, idx, E), axis=1, keepdims=True)
    pm = jnp.where(idx == i1, -jnp.inf, probs)
    p2 = jnp.max(pm, axis=1, keepdims=True)
    i2 = jnp.min(jnp.where(pm == p2, idx, E), axis=1, keepdims=True)
    t = jnp.exp(p2 - p1)
    q1 = 1.0 / (1.0 + t)
    q2 = t / (1.0 + t)

    i12 = jnp.concatenate([i1, i2], axis=0)
    qpair_ref[...] = jnp.concatenate([q1, q2], axis=0)
    lane8 = lax.broadcasted_iota(jnp.int32, (P, E), 1)
    oh = (i12 == lane8).astype(jnp.float32)
    oh_ref[...] = oh

    cnt = jnp.sum(oh, axis=0, keepdims=True)
    npad = jnp.ceil(cnt / BK) * BK
    r8 = lax.broadcasted_iota(jnp.int32, (E, E), 0)
    c8 = lax.broadcasted_iota(jnp.int32, (E, E), 1)
    excl8 = (r8 < c8).astype(jnp.float32)
    poff = jnp.dot(npad, excl8, preferred_element_type=jnp.float32)

    rr = lax.broadcasted_iota(jnp.int32, (CH, CH), 0)
    cc = lax.broadcasted_iota(jnp.int32, (CH, CH), 1)
    ltri = (rr > cc).astype(jnp.float32)

    def body(i, base):
        blk = oh_ref[pl.ds(i * CH, CH), :]
        rank = jnp.dot(ltri, blk, preferred_element_type=jnp.float32)
        row = jnp.sum((rank + base + poff) * blk, axis=1, keepdims=True)
        pos_ref[pl.ds(i * CH, CH), :] = row.astype(jnp.int32)
        return base + jnp.sum(blk, axis=0, keepdims=True)

    lax.fori_loop(0, P // CH, body, jnp.zeros((1, E), jnp.float32))

    nblk = npad / BK
    blkoff = jnp.dot(nblk, excl8, preferred_element_type=jnp.float32)
    biota = lax.broadcasted_iota(jnp.int32, (32, E), 0).astype(jnp.float32)
    ge = (biota >= blkoff).astype(jnp.float32)
    beid_ref[...] = (jnp.sum(ge, axis=1, keepdims=True) - 1.0).astype(jnp.int32)


PPT = P // NS
ZPT = NBK // NS


def _sc_dispatch(pos_hbm, q_hbm, x_hbm, xg_hbm, wr_hbm,
                 myidx, tokv, qv, zbi, zbf, idx_v, buf, rt_sh, wr_sh, sem):
    cid = lax.axis_index("c")
    sid = lax.axis_index("s")
    pb = sid * PPT

    for j in range(2):
        pltpu.sync_copy(pos_hbm.at[pl.ds(pb + j * 128, 128)], myidx.at[j])
        pltpu.sync_copy(q_hbm.at[pl.ds(pb + j * 128, 128)], qv.at[j])

    def tbody(i, _):
        tokv[0, pl.ds(i * 16, 16)] = (lax.iota(jnp.int32, 16)
                                      + (pb + i * 16)) & (S - 1)
        tokv[1, pl.ds(i * 16, 16)] = (lax.iota(jnp.int32, 16)
                                      + (pb + 128 + i * 16)) & (S - 1)
        return 0

    lax.fori_loop(0, 8, tbody, 0)

    zi = jnp.zeros((16,), jnp.int32)
    zf = jnp.zeros((16,), jnp.float32)

    def zbody(i, _):
        zbi[pl.ds(i * 16, 16)] = zi
        zbf[pl.ds(i * 16, 16)] = zf
        return 0

    lax.fori_loop(0, ZPT // 16, zbody, 0)
    pltpu.sync_copy(zbi, rt_sh.at[pl.ds(sid * ZPT, ZPT)])
    pltpu.sync_copy(zbf, wr_sh.at[pl.ds(sid * ZPT, ZPT)])
    plsc.subcore_barrier()
    for j in range(2):
        pltpu.sync_copy(tokv.at[j], rt_sh.at[myidx.at[j]], add=True)
        pltpu.sync_copy(qv.at[j], wr_sh.at[myidx.at[j]], add=True)
    plsc.subcore_barrier()

    @pl.when(cid == 0)
    def _():
        pltpu.sync_copy(wr_sh.at[pl.ds(sid * ZPT, ZPT)], zbf)
        pltpu.sync_copy(zbf, wr_hbm.at[pl.ds(sid * ZPT, ZPT)])

    tid = cid * NS + sid
    base = tid * RPT
    pltpu.sync_copy(rt_sh.at[pl.ds(base, RPT)], idx_v)
    pltpu.async_copy(x_hbm.at[idx_v.at[pl.ds(0, C1)]], buf, sem).wait()
    pltpu.sync_copy(buf, xg_hbm.at[pl.ds(base, C1)])
    pltpu.async_copy(x_hbm.at[idx_v.at[pl.ds(C1, C2)]],
                     buf.at[pl.ds(0, C2)], sem).wait()
    pltpu.sync_copy(buf.at[pl.ds(0, C2)], xg_hbm.at[pl.ds(base + C1, C2)])


def _gmm_kernel(beid_ref, xg_ref, w1_ref, b1_ref, w2_ref, b2_ref, wr_ref,
                y_ref):
    del beid_ref
    xv = xg_ref[...]
    acc = jnp.zeros((BK, D), jnp.float32)
    for hc in range(2):
        sl = slice(hc * (H // 2), (hc + 1) * (H // 2))
        h = jnp.maximum(
            jnp.dot(xv, w1_ref[0, :, sl], preferred_element_type=jnp.float32)
            + b1_ref[0, :, sl], 0.0)
        acc = acc + jnp.dot(h, w2_ref[0, sl, :],
                            preferred_element_type=jnp.float32)
    y_ref[...] = (acc + b2_ref[0]) * wr_ref[...]


def _sc_combine(pos_hbm, y_hbm, out_hbm, idx1_v, idx2_v, buf1, buf2, sem):
    cid = lax.axis_index("c")
    sid = lax.axis_index("s")
    tid = cid * NS + sid
    tb = tid * TPB
    pltpu.sync_copy(pos_hbm.at[pl.ds(tb, TPB)], idx1_v)
    pltpu.sync_copy(pos_hbm.at[pl.ds(S + tb, TPB)], idx2_v)
    c1 = pltpu.async_copy(y_hbm.at[idx1_v], buf1, sem)
    c2 = pltpu.async_copy(y_hbm.at[idx2_v], buf2, sem)
    c1.wait()
    c2.wait()

    def rbody(r, _):
        for c in range(D // 16):
            sl = pl.ds(c * 16, 16)
            buf1[r, sl] = buf1[r, sl] + buf2[r, sl]
        return 0

    lax.fori_loop(0, TPB, rbody, 0)
    pltpu.sync_copy(buf1, out_hbm.at[pl.ds(tb, TPB)])


@functools.lru_cache(maxsize=None)
def _sc_calls():
    mesh = plsc.VectorSubcoreMesh(core_axis_name="c", subcore_axis_name="s",
                                  num_cores=NC, num_subcores=NS)
    dispatch = pl.kernel(
        _sc_dispatch,
        out_type=[jax.ShapeDtypeStruct((NBK, D), jnp.float32),
                  jax.ShapeDtypeStruct((NBK,), jnp.float32)],
        mesh=mesh,
        scratch_types=[
            pltpu.VMEM((2, 128), jnp.int32),
            pltpu.VMEM((2, 128), jnp.int32),
            pltpu.VMEM((2, 128), jnp.float32),
            pltpu.VMEM((ZPT,), jnp.int32),
            pltpu.VMEM((ZPT,), jnp.float32),
            pltpu.VMEM((RPT,), jnp.int32),
            pltpu.VMEM((C1, D), jnp.float32),
            pltpu.VMEM_SHARED((NBK,), jnp.int32),
            pltpu.VMEM_SHARED((NBK,), jnp.float32),
            pltpu.SemaphoreType.DMA,
        ],
        compiler_params=pltpu.CompilerParams(needs_layout_passes=False),
    )
    combine = pl.kernel(
        _sc_combine,
        out_type=jax.ShapeDtypeStruct((S, D), jnp.float32),
        mesh=mesh,
        scratch_types=[
            pltpu.VMEM((TPB,), jnp.int32),
            pltpu.VMEM((TPB,), jnp.int32),
            pltpu.VMEM((TPB, D), jnp.float32),
            pltpu.VMEM((TPB, D), jnp.float32),
            pltpu.SemaphoreType.DMA,
        ],
        compiler_params=pltpu.CompilerParams(needs_layout_passes=False),
    )
    return dispatch, combine


@jax.jit
def kernel(x, Wg, bg, W1, b1, W2, b2):
    B = x.shape[0]
    xs = x.reshape(S, D)

    gate, pos, qpair, beid = pl.pallas_call(
        _router_kernel,
        out_shape=[
            jax.ShapeDtypeStruct((S, E), jnp.float32),
            jax.ShapeDtypeStruct((P, 1), jnp.int32),
            jax.ShapeDtypeStruct((P, 1), jnp.float32),
            jax.ShapeDtypeStruct((32, 1), jnp.int32),
        ],
        scratch_shapes=[pltpu.VMEM((P, E), jnp.float32)],
    )(xs, Wg, bg.reshape(1, E))

    dispatch_call, combine_call = _sc_calls()
    pos1 = pos.reshape(P)
    xg, wrow = dispatch_call(pos1, qpair.reshape(P), xs)

    y = pl.pallas_call(
        _gmm_kernel,
        grid_spec=pltpu.PrefetchScalarGridSpec(
            num_scalar_prefetch=1,
            grid=(NB,),
            in_specs=[
                pl.BlockSpec((BK, D), lambda i, beid: (i, 0)),
                pl.BlockSpec((1, D, H), lambda i, beid: (beid[i], 0, 0)),
                pl.BlockSpec((1, 1, H), lambda i, beid: (beid[i], 0, 0)),
                pl.BlockSpec((1, H, D), lambda i, beid: (beid[i], 0, 0)),
                pl.BlockSpec((1, 1, D), lambda i, beid: (beid[i], 0, 0)),
                pl.BlockSpec((BK, 1), lambda i, beid: (i, 0)),
            ],
            out_specs=pl.BlockSpec((BK, D), lambda i, beid: (i, 0)),
        ),
        out_shape=jax.ShapeDtypeStruct((NBK, D), jnp.float32),
    )(beid.reshape(32), xg, W1, b1.reshape(E, 1, H), W2, b2.reshape(E, 1, D),
      wrow.reshape(NBK, 1))

    out = combine_call(pos1, y)
    return out.reshape(B, S, D), gate.reshape(B, S, E)


# speedup vs baseline: 1.0088x; 1.0014x over previous
import functools

import jax
import jax.numpy as jnp
from jax import lax
from jax.experimental import pallas as pl
from jax.experimental.pallas import tpu as pltpu
from jax.experimental.pallas import tpu_sc as plsc

S, D = 2048, 768
E, K, H = 8, 2, 1536
P = K * S
BK = 256
NB = P // BK + (E - 1)
NBK = NB * BK
NC, NS = 2, 16
NW = NC * NS
RPT = NBK // NW
C1, C2 = 96, 88
TPB = S // NW
CH = 128


def _router_kernel(x_ref, wg_ref, bg_ref, gate_ref, pos_ref, qpair_ref,
                   beid_ref, oh_ref):
    xv = x_ref[...]
    logits = jnp.dot(xv, wg_ref[...], preferred_element_type=jnp.float32)
    logits = logits + bg_ref[...]
    m = jnp.max(logits, axis=1, keepdims=True)
    ex = jnp.exp(logits - m)
    probs = ex / jnp.sum(ex, axis=1, keepdims=True)
    gate_ref[...] = probs

    idx = lax.broadcasted_iota(jnp.int32, (S, E), 1)
    p1 = jnp.max(probs, axis=1, keepdims=True)
    i1 = jnp.min(jnp.where(probs == p1, idx, E), axis=1, keepdims=True)
    pm = jnp.where(idx == i1, -jnp.inf, probs)
    p2 = jnp.max(pm, axis=1, keepdims=True)
    i2 = jnp.min(jnp.where(pm == p2, idx, E), axis=1, keepdims=True)
    t = jnp.exp(p2 - p1)
    q1 = 1.0 / (1.0 + t)
    q2 = t / (1.0 + t)

    i12 = jnp.concatenate([i1, i2], axis=0)
    qpair_ref[...] = jnp.concatenate([q1, q2], axis=0)
    lane8 = lax.broadcasted_iota(jnp.int32, (P, E), 1)
    oh = (i12 == lane8).astype(jnp.float32)
    oh_ref[...] = oh

    cnt = jnp.sum(oh, axis=0, keepdims=True)
    npad = jnp.ceil(cnt / BK) * BK
    r8 = lax.broadcasted_iota(jnp.int32, (E, E), 0)
    c8 = lax.broadcasted_iota(jnp.int32, (E, E), 1)
    excl8 = (r8 < c8).astype(jnp.float32)
    poff = jnp.dot(npad, excl8, preferred_element_type=jnp.float32)

    rr = lax.broadcasted_iota(jnp.int32, (CH, CH), 0)
    cc = lax.broadcasted_iota(jnp.int32, (CH, CH), 1)
    ltri = (rr > cc).astype(jnp.float32)

    def body(i, base):
        blk = oh_ref[pl.ds(i * CH, CH), :]
        rank = jnp.dot(ltri, blk, preferred_element_type=jnp.float32)
        row = jnp.sum((rank + base + poff) * blk, axis=1, keepdims=True)
        pos_ref[pl.ds(i * CH, CH), :] = row.astype(jnp.int32)
        return base + jnp.sum(blk, axis=0, keepdims=True)

    lax.fori_loop(0, P // CH, body, jnp.zeros((1, E), jnp.float32))

    nblk = npad / BK
    blkoff = jnp.dot(nblk, excl8, preferred_element_type=jnp.float32)
    biota = lax.broadcasted_iota(jnp.int32, (32, E), 0).astype(jnp.float32)
    ge = (biota >= blkoff).astype(jnp.float32)
    beid_ref[...] = (jnp.sum(ge, axis=1, keepdims=True) - 1.0).astype(jnp.int32)


PPT = P // NS
ZPT = NBK // NS


def _sc_dispatch(pos_hbm, q_hbm, x_hbm, xg_hbm, wr_hbm,
                 myidx, tokv, qv, zbi, zbf, idx_v, buf, rt_sh, wr_sh, sem):
    cid = lax.axis_index("c")
    sid = lax.axis_index("s")
    pb = sid * PPT

    with jax.named_scope("disp_load"):
        for j in range(2):
            pltpu.sync_copy(pos_hbm.at[pl.ds(pb + j * 128, 128)], myidx.at[j])
            pltpu.sync_copy(q_hbm.at[pl.ds(pb + j * 128, 128)], qv.at[j])

    def tbody(i, _):
        tokv[0, pl.ds(i * 16, 16)] = (lax.iota(jnp.int32, 16)
                                      + (pb + i * 16)) & (S - 1)
        tokv[1, pl.ds(i * 16, 16)] = (lax.iota(jnp.int32, 16)
                                      + (pb + 128 + i * 16)) & (S - 1)
        return 0

    lax.fori_loop(0, 8, tbody, 0)

    zi = jnp.zeros((16,), jnp.int32)
    zf = jnp.zeros((16,), jnp.float32)

    def zbody(i, _):
        zbi[pl.ds(i * 16, 16)] = zi
        zbf[pl.ds(i * 16, 16)] = zf
        return 0

    with jax.named_scope("disp_zero"):
        lax.fori_loop(0, ZPT // 16, zbody, 0)
        pltpu.sync_copy(zbi, rt_sh.at[pl.ds(sid * ZPT, ZPT)])
        pltpu.sync_copy(zbf, wr_sh.at[pl.ds(sid * ZPT, ZPT)])
        plsc.subcore_barrier()
    with jax.named_scope("disp_scatter"):
        for j in range(2):
            pltpu.sync_copy(tokv.at[j], rt_sh.at[myidx.at[j]], add=True)
            pltpu.sync_copy(qv.at[j], wr_sh.at[myidx.at[j]], add=True)
        plsc.subcore_barrier()

    @pl.when(cid == 0)
    def _():
        pltpu.sync_copy(wr_sh.at[pl.ds(sid * ZPT, ZPT)], zbf)
        pltpu.sync_copy(zbf, wr_hbm.at[pl.ds(sid * ZPT, ZPT)])

    with jax.named_scope("disp_gather"):
        tid = cid * NS + sid
        base = tid * RPT
        pltpu.sync_copy(rt_sh.at[pl.ds(base, RPT)], idx_v)
        pltpu.async_copy(x_hbm.at[idx_v.at[pl.ds(0, C1)]], buf, sem).wait()
        pltpu.sync_copy(buf, xg_hbm.at[pl.ds(base, C1)])
        pltpu.async_copy(x_hbm.at[idx_v.at[pl.ds(C1, C2)]],
                         buf.at[pl.ds(0, C2)], sem).wait()
        pltpu.sync_copy(buf.at[pl.ds(0, C2)], xg_hbm.at[pl.ds(base + C1, C2)])


def _gmm_kernel(beid_ref, xg_ref, w1_ref, b1_ref, w2_ref, b2_ref, wr_ref,
                y_ref):
    del beid_ref
    xv = xg_ref[...]
    acc = jnp.zeros((BK, D), jnp.float32)
    for hc in range(2):
        sl = slice(hc * (H // 2), (hc + 1) * (H // 2))
        h = jnp.maximum(
            jnp.dot(xv, w1_ref[0, :, sl], preferred_element_type=jnp.float32)
            + b1_ref[0, :, sl], 0.0)
        acc = acc + jnp.dot(h, w2_ref[0, sl, :],
                            preferred_element_type=jnp.float32)
    y_ref[...] = (acc + b2_ref[0]) * wr_ref[...]


def _sc_combine(pos_hbm, y_hbm, out_hbm, idx1_v, idx2_v, buf1, buf2, sem):
    cid = lax.axis_index("c")
    sid = lax.axis_index("s")
    tid = cid * NS + sid
    tb = tid * TPB
    pltpu.sync_copy(pos_hbm.at[pl.ds(tb, TPB)], idx1_v)
    pltpu.sync_copy(pos_hbm.at[pl.ds(S + tb, TPB)], idx2_v)
    c1 = pltpu.async_copy(y_hbm.at[idx1_v], buf1, sem)
    c2 = pltpu.async_copy(y_hbm.at[idx2_v], buf2, sem)
    c1.wait()
    c2.wait()

    def rbody(r, _):
        for c in range(D // 16):
            sl = pl.ds(c * 16, 16)
            buf1[r, sl] = buf1[r, sl] + buf2[r, sl]
        return 0

    lax.fori_loop(0, TPB, rbody, 0)
    pltpu.sync_copy(buf1, out_hbm.at[pl.ds(tb, TPB)])


@functools.lru_cache(maxsize=None)
def _sc_calls():
    mesh = plsc.VectorSubcoreMesh(core_axis_name="c", subcore_axis_name="s",
                                  num_cores=NC, num_subcores=NS)
    dispatch = pl.kernel(
        _sc_dispatch,
        out_type=[jax.ShapeDtypeStruct((NBK, D), jnp.float32),
                  jax.ShapeDtypeStruct((NBK,), jnp.float32)],
        mesh=mesh,
        scratch_types=[
            pltpu.VMEM((2, 128), jnp.int32),
            pltpu.VMEM((2, 128), jnp.int32),
            pltpu.VMEM((2, 128), jnp.float32),
            pltpu.VMEM((ZPT,), jnp.int32),
            pltpu.VMEM((ZPT,), jnp.float32),
            pltpu.VMEM((RPT,), jnp.int32),
            pltpu.VMEM((C1, D), jnp.float32),
            pltpu.VMEM_SHARED((NBK,), jnp.int32),
            pltpu.VMEM_SHARED((NBK,), jnp.float32),
            pltpu.SemaphoreType.DMA,
        ],
        compiler_params=pltpu.CompilerParams(needs_layout_passes=False),
    )
    combine = pl.kernel(
        _sc_combine,
        out_type=jax.ShapeDtypeStruct((S, D), jnp.float32),
        mesh=mesh,
        scratch_types=[
            pltpu.VMEM((TPB,), jnp.int32),
            pltpu.VMEM((TPB,), jnp.int32),
            pltpu.VMEM((TPB, D), jnp.float32),
            pltpu.VMEM((TPB, D), jnp.float32),
            pltpu.SemaphoreType.DMA,
        ],
        compiler_params=pltpu.CompilerParams(needs_layout_passes=False),
    )
    return dispatch, combine


@jax.jit
def kernel(x, Wg, bg, W1, b1, W2, b2):
    B = x.shape[0]
    xs = x.reshape(S, D)

    gate, pos, qpair, beid = pl.pallas_call(
        _router_kernel,
        out_shape=[
            jax.ShapeDtypeStruct((S, E), jnp.float32),
            jax.ShapeDtypeStruct((P, 1), jnp.int32),
            jax.ShapeDtypeStruct((P, 1), jnp.float32),
            jax.ShapeDtypeStruct((32, 1), jnp.int32),
        ],
        scratch_shapes=[pltpu.VMEM((P, E), jnp.float32)],
    )(xs, Wg, bg.reshape(1, E))

    dispatch_call, combine_call = _sc_calls()
    pos1 = pos.reshape(P)
    xg, wrow = dispatch_call(pos1, qpair.reshape(P), xs)

    y = pl.pallas_call(
        _gmm_kernel,
        grid_spec=pltpu.PrefetchScalarGridSpec(
            num_scalar_prefetch=1,
            grid=(NB,),
            in_specs=[
                pl.BlockSpec((BK, D), lambda i, beid: (i, 0)),
                pl.BlockSpec((1, D, H), lambda i, beid: (beid[i], 0, 0)),
                pl.BlockSpec((1, 1, H), lambda i, beid: (beid[i], 0, 0)),
                pl.BlockSpec((1, H, D), lambda i, beid: (beid[i], 0, 0)),
                pl.BlockSpec((1, 1, D), lambda i, beid: (beid[i], 0, 0)),
                pl.BlockSpec((BK, 1), lambda i, beid: (i, 0)),
            ],
            out_specs=pl.BlockSpec((BK, D), lambda i, beid: (i, 0)),
        ),
        out_shape=jax.ShapeDtypeStruct((NBK, D), jnp.float32),
    )(beid.reshape(32), xg, W1, b1.reshape(E, 1, H), W2, b2.reshape(E, 1, D),
      wrow.reshape(NBK, 1))

    out = combine_call(pos1, y)
    return out.reshape(B, S, D), gate.reshape(B, S, E)


# speedup vs baseline: 1.6313x; 1.6170x over previous
import functools

import jax
import jax.numpy as jnp
from jax import lax
from jax.experimental import pallas as pl
from jax.experimental.pallas import tpu as pltpu
from jax.experimental.pallas import tpu_sc as plsc

S, D = 2048, 768
E, K, H = 8, 2, 1536
P = K * S
BK = 128
NB = P // BK + (E - 1)
NBK = NB * BK
NBKD = 5120
NBPAD = 40
NC, NS = 2, 16
NW = NC * NS
RPT = NBKD // NW
C1, C2 = 80, 80
TPB = S // NW
CH = 512


def _router_kernel(x_ref, wg_ref, bg_ref, gate_ref, pos_ref, qpair_ref,
                   beid_ref, oh_ref):
    xv = x_ref[...]
    logits = jnp.dot(xv, wg_ref[...], preferred_element_type=jnp.float32)
    logits = logits + bg_ref[...]
    m = jnp.max(logits, axis=1, keepdims=True)
    ex = jnp.exp(logits - m)
    probs = ex / jnp.sum(ex, axis=1, keepdims=True)
    gate_ref[...] = probs

    idx = lax.broadcasted_iota(jnp.int32, (S, E), 1)
    p1 = jnp.max(probs, axis=1, keepdims=True)
    i1 = jnp.min(jnp.where(probs == p1, idx, E), axis=1, keepdims=True)
    pm = jnp.where(idx == i1, -jnp.inf, probs)
    p2 = jnp.max(pm, axis=1, keepdims=True)
    i2 = jnp.min(jnp.where(pm == p2, idx, E), axis=1, keepdims=True)
    t = jnp.exp(p2 - p1)
    q1 = 1.0 / (1.0 + t)
    q2 = t / (1.0 + t)

    i12 = jnp.concatenate([i1, i2], axis=0)
    qpair_ref[...] = jnp.concatenate([q1, q2], axis=0)
    lane8 = lax.broadcasted_iota(jnp.int32, (P, E), 1)
    oh = (i12 == lane8).astype(jnp.float32)
    oh_ref[...] = oh

    cnt = jnp.sum(oh, axis=0, keepdims=True)
    npad = jnp.ceil(cnt / BK) * BK
    r8 = lax.broadcasted_iota(jnp.int32, (E, E), 0)
    c8 = lax.broadcasted_iota(jnp.int32, (E, E), 1)
    excl8 = (r8 < c8).astype(jnp.float32)
    poff = jnp.dot(npad, excl8, preferred_element_type=jnp.float32)

    rr = lax.broadcasted_iota(jnp.int32, (CH, CH), 0)
    cc = lax.broadcasted_iota(jnp.int32, (CH, CH), 1)
    ltri = (rr > cc).astype(jnp.float32)

    def body(i, base):
        blk = oh_ref[pl.ds(i * CH, CH), :]
        rank = jnp.dot(ltri, blk, preferred_element_type=jnp.float32)
        row = jnp.sum((rank + base + poff) * blk, axis=1, keepdims=True)
        pos_ref[pl.ds(i * CH, CH), :] = row.astype(jnp.int32)
        return base + jnp.sum(blk, axis=0, keepdims=True)

    lax.fori_loop(0, P // CH, body, jnp.zeros((1, E), jnp.float32))

    nblk = npad / BK
    blkoff = jnp.dot(nblk, excl8, preferred_element_type=jnp.float32)
    biota = lax.broadcasted_iota(jnp.int32, (NBPAD, E), 0).astype(jnp.float32)
    ge = (biota >= blkoff).astype(jnp.float32)
    beid_ref[...] = (jnp.sum(ge, axis=1, keepdims=True) - 1.0).astype(jnp.int32)


PPT = P // NS
ZPT = NBKD // NS
ZPT16 = ZPT


def _sc_dispatch(pos_hbm, q_hbm, x_hbm, xg_hbm, wr_hbm,
                 myidx, tokv, qv, zbi, zbf, idx_v, buf_a, buf_b, rt_sh, wr_sh,
                 sga, sgb, swa, swb):
    cid = lax.axis_index("c")
    sid = lax.axis_index("s")
    pb = sid * PPT

    with jax.named_scope("disp_load"):
        for j in range(2):
            pltpu.sync_copy(pos_hbm.at[pl.ds(pb + j * 128, 128)], myidx.at[j])
            pltpu.sync_copy(q_hbm.at[pl.ds(pb + j * 128, 128)], qv.at[j])

    def tbody(i, _):
        for j in range(2):
            sl = pl.ds(i * 16, 16)
            tok = (lax.iota(jnp.int32, 16) + (pb + j * 128 + i * 16)) & (S - 1)
            tokv[j, sl] = tok - (myidx[j, sl] & (S - 1))
        return 0

    lax.fori_loop(0, 8, tbody, 0)

    zf = jnp.zeros((16,), jnp.float32)
    zb = sid * ZPT

    def zbody(i, _):
        zbi[pl.ds(i * 16, 16)] = (lax.iota(jnp.int32, 16) + (zb + i * 16)) & (S - 1)
        zbf[pl.ds(i * 16, 16)] = zf
        return 0

    with jax.named_scope("disp_zero"):
        lax.fori_loop(0, ZPT16 // 16, zbody, 0)
        pltpu.sync_copy(zbi.at[pl.ds(0, ZPT)], rt_sh.at[pl.ds(zb, ZPT)])
        pltpu.sync_copy(zbf.at[pl.ds(0, ZPT)], wr_sh.at[pl.ds(zb, ZPT)])
        plsc.subcore_barrier()
    with jax.named_scope("disp_scatter"):
        for j in range(2):
            pltpu.sync_copy(tokv.at[j], rt_sh.at[myidx.at[j]], add=True)
            pltpu.sync_copy(qv.at[j], wr_sh.at[myidx.at[j]], add=True)
        plsc.subcore_barrier()

    @pl.when(cid == 0)
    def _():
        pltpu.sync_copy(wr_sh.at[pl.ds(zb, ZPT)], zbf.at[pl.ds(0, ZPT)])
        pltpu.sync_copy(zbf.at[pl.ds(0, ZPT)], wr_hbm.at[pl.ds(zb, ZPT)])

    with jax.named_scope("disp_gather"):
        tid = cid * NS + sid
        base = tid * RPT
        pltpu.sync_copy(rt_sh.at[pl.ds(base, RPT)], idx_v)
        g1 = pltpu.async_copy(x_hbm.at[idx_v.at[pl.ds(0, C1)]], buf_a, sga)
        g2 = pltpu.async_copy(x_hbm.at[idx_v.at[pl.ds(C1, C2)]], buf_b, sgb)
        g1.wait()
        w1 = pltpu.async_copy(buf_a, xg_hbm.at[pl.ds(base, C1)], swa)
        g2.wait()
        w2 = pltpu.async_copy(buf_b, xg_hbm.at[pl.ds(base + C1, C2)], swb)
        w1.wait()
        w2.wait()


def _gmm_kernel(beid_ref, xg_ref, w1_ref, b1_ref, w2_ref, b2_ref, wr_ref,
                y_ref):
    del beid_ref
    xv = xg_ref[...]
    acc = jnp.zeros((BK, D), jnp.float32)
    for hc in range(2):
        sl = slice(hc * (H // 2), (hc + 1) * (H // 2))
        h = jnp.maximum(
            jnp.dot(xv, w1_ref[0, :, sl], preferred_element_type=jnp.float32)
            + b1_ref[0, :, sl], 0.0)
        acc = acc + jnp.dot(h, w2_ref[0, sl, :],
                            preferred_element_type=jnp.float32)
    y_ref[...] = (acc + b2_ref[0]) * wr_ref[...]


def _sc_combine(pos_hbm, y_hbm, out_hbm, idx1_v, idx2_v, buf1, buf2, sem):
    cid = lax.axis_index("c")
    sid = lax.axis_index("s")
    tid = cid * NS + sid
    tb = tid * TPB
    pltpu.sync_copy(pos_hbm.at[pl.ds(tb, TPB)], idx1_v)
    pltpu.sync_copy(pos_hbm.at[pl.ds(S + tb, TPB)], idx2_v)
    c1 = pltpu.async_copy(y_hbm.at[idx1_v], buf1, sem)
    c2 = pltpu.async_copy(y_hbm.at[idx2_v], buf2, sem)
    c1.wait()
    c2.wait()

    def rbody(r, _):
        for c in range(D // 16):
            sl = pl.ds(c * 16, 16)
            buf1[r, sl] = buf1[r, sl] + buf2[r, sl]
        return 0

    lax.fori_loop(0, TPB, rbody, 0)
    pltpu.sync_copy(buf1, out_hbm.at[pl.ds(tb, TPB)])


@functools.lru_cache(maxsize=None)
def _sc_calls():
    mesh = plsc.VectorSubcoreMesh(core_axis_name="c", subcore_axis_name="s",
                                  num_cores=NC, num_subcores=NS)
    dispatch = pl.kernel(
        _sc_dispatch,
        out_type=[jax.ShapeDtypeStruct((NBKD, D), jnp.float32),
                  jax.ShapeDtypeStruct((NBKD,), jnp.float32)],
        mesh=mesh,
        scratch_types=[
            pltpu.VMEM((2, 128), jnp.int32),
            pltpu.VMEM((2, 128), jnp.int32),
            pltpu.VMEM((2, 128), jnp.float32),
            pltpu.VMEM((ZPT16,), jnp.int32),
            pltpu.VMEM((ZPT16,), jnp.float32),
            pltpu.VMEM((RPT,), jnp.int32),
            pltpu.VMEM((C1, D), jnp.float32),
            pltpu.VMEM((C2, D), jnp.float32),
            pltpu.VMEM_SHARED((NBKD,), jnp.int32),
            pltpu.VMEM_SHARED((NBKD,), jnp.float32),
            pltpu.SemaphoreType.DMA,
            pltpu.SemaphoreType.DMA,
            pltpu.SemaphoreType.DMA,
            pltpu.SemaphoreType.DMA,
        ],
        compiler_params=pltpu.CompilerParams(needs_layout_passes=False),
    )
    combine = pl.kernel(
        _sc_combine,
        out_type=jax.ShapeDtypeStruct((S, D), jnp.float32),
        mesh=mesh,
        scratch_types=[
            pltpu.VMEM((TPB,), jnp.int32),
            pltpu.VMEM((TPB,), jnp.int32),
            pltpu.VMEM((TPB, D), jnp.float32),
            pltpu.VMEM((TPB, D), jnp.float32),
            pltpu.SemaphoreType.DMA,
        ],
        compiler_params=pltpu.CompilerParams(needs_layout_passes=False),
    )
    return dispatch, combine


@jax.jit
def kernel(x, Wg, bg, W1, b1, W2, b2):
    B = x.shape[0]
    xs = x.reshape(S, D)

    gate, pos, qpair, beid = pl.pallas_call(
        _router_kernel,
        out_shape=[
            jax.ShapeDtypeStruct((S, E), jnp.float32),
            jax.ShapeDtypeStruct((P, 1), jnp.int32),
            jax.ShapeDtypeStruct((P, 1), jnp.float32),
            jax.ShapeDtypeStruct((NBPAD, 1), jnp.int32),
        ],
        scratch_shapes=[pltpu.VMEM((P, E), jnp.float32)],
    )(xs, Wg, bg.reshape(1, E))

    dispatch_call, combine_call = _sc_calls()
    pos1 = pos.reshape(P)
    xg, wrow = dispatch_call(pos1, qpair.reshape(P), xs)

    y = pl.pallas_call(
        _gmm_kernel,
        grid_spec=pltpu.PrefetchScalarGridSpec(
            num_scalar_prefetch=1,
            grid=(NB,),
            in_specs=[
                pl.BlockSpec((BK, D), lambda i, beid: (i, 0)),
                pl.BlockSpec((1, D, H), lambda i, beid: (beid[i], 0, 0)),
                pl.BlockSpec((1, 1, H), lambda i, beid: (beid[i], 0, 0)),
                pl.BlockSpec((1, H, D), lambda i, beid: (beid[i], 0, 0)),
                pl.BlockSpec((1, 1, D), lambda i, beid: (beid[i], 0, 0)),
                pl.BlockSpec((BK, 1), lambda i, beid: (i, 0)),
            ],
            out_specs=pl.BlockSpec((BK, D), lambda i, beid: (i, 0)),
        ),
        out_shape=jax.ShapeDtypeStruct((NBK, D), jnp.float32),
    )(beid.reshape(NBPAD), xg, W1, b1.reshape(E, 1, H), W2,
      b2.reshape(E, 1, D), wrow.reshape(NBKD, 1))

    out = combine_call(pos1, y)
    return out.reshape(B, S, D), gate.reshape(B, S, E)
